# Initial kernel scaffold; baseline (speedup 1.0000x reference)
import jax
import jax.numpy as jnp
from jax.experimental import pallas as pl


def kernel(atom_types, table):
    raise NotImplementedError("write your pallas kernel here")


# speedup vs baseline: 4.0813x; 4.0813x over previous
import functools

import jax
import jax.numpy as jnp
from jax import lax
from jax.experimental import pallas as pl
from jax.experimental.pallas import tpu as pltpu
from jax.experimental.pallas import tpu_sc as plsc

EMB_D = 64


@functools.lru_cache(maxsize=None)
def _make_gather(B: int, V: int):
    info = plsc.get_sparse_core_info()
    NC, NS = info.num_cores, info.num_subcores
    NW = NC * NS
    assert B % NW == 0
    b_per_w = B // NW
    C = 800
    assert b_per_w % C == 0
    n_chunks = b_per_w // C
    mesh = plsc.VectorSubcoreMesh(core_axis_name="c", subcore_axis_name="s")

    @functools.partial(
        pl.kernel,
        mesh=mesh,
        compiler_params=pltpu.CompilerParams(use_tc_tiling_on_sc=False),
        out_type=jax.ShapeDtypeStruct((B, EMB_D), jnp.float32),
        scratch_types=[
            pltpu.VMEM((C,), jnp.int32),
            pltpu.VMEM((C, EMB_D), jnp.float32),
            pltpu.SemaphoreType.DMA,
        ],
    )
    def k(idx_hbm, table_hbm, out_hbm, idx_v, rows_v, sem):
        wid = lax.axis_index("s") * NC + lax.axis_index("c")
        base = wid * b_per_w

        def body(j, carry):
            off = base + j * C
            pltpu.sync_copy(idx_hbm.at[pl.ds(off, C)], idx_v)
            pltpu.async_copy(table_hbm.at[idx_v], rows_v, sem).wait()
            pltpu.sync_copy(rows_v, out_hbm.at[pl.ds(off, C)])
            return carry

        lax.fori_loop(0, n_chunks, body, 0)

    return k


@jax.jit
def kernel(atom_types, table):
    n_mol, n_atom = atom_types.shape
    B = n_mol * n_atom
    flat = atom_types.reshape(B).astype(jnp.int32)
    out = _make_gather(B, table.shape[0])(flat, table)
    return out.reshape(n_mol, n_atom, EMB_D)


# speedup vs baseline: 4.2662x; 1.0453x over previous
import functools

import jax
import jax.numpy as jnp
from jax import lax
from jax.experimental import pallas as pl
from jax.experimental.pallas import tpu as pltpu
from jax.experimental.pallas import tpu_sc as plsc

EMB_D = 64
NBUF = 4
G = 2
C = 400


@functools.lru_cache(maxsize=None)
def _make_gather(B: int, V: int):
    info = plsc.get_sparse_core_info()
    NC, NS = info.num_cores, info.num_subcores
    NW = NC * NS
    assert B % NW == 0
    b_per_w = B // NW
    assert b_per_w % (C * NBUF) == 0
    n_chunks = b_per_w // C
    n_pass = n_chunks // NBUF
    mesh = plsc.VectorSubcoreMesh(core_axis_name="c", subcore_axis_name="s")

    scratch = (
        [pltpu.VMEM((C,), jnp.int32) for _ in range(NBUF)]
        + [pltpu.VMEM((C, EMB_D), jnp.float32) for _ in range(NBUF)]
        + [pltpu.SemaphoreType.DMA for _ in range(2 * NBUF)]
    )

    @functools.partial(
        pl.kernel,
        mesh=mesh,
        compiler_params=pltpu.CompilerParams(use_tc_tiling_on_sc=False),
        out_type=jax.ShapeDtypeStruct((B, EMB_D), jnp.float32),
        scratch_types=scratch,
    )
    def k(idx_hbm, table_hbm, out_hbm, *rest):
        idx_bufs = rest[0:NBUF]
        row_bufs = rest[NBUF:2 * NBUF]
        sg = rest[2 * NBUF:3 * NBUF]
        so = rest[3 * NBUF:4 * NBUF]
        wid = lax.axis_index("s") * NC + lax.axis_index("c")
        base = wid * b_per_w

        def start_gather(j, b):
            off = base + j * C
            pltpu.sync_copy(idx_hbm.at[pl.ds(off, C)], idx_bufs[b])
            pltpu.async_copy(table_hbm.at[idx_bufs[b]], row_bufs[b], sg[b])

        def wait_gather(b):
            pltpu.make_async_copy(
                table_hbm.at[idx_bufs[b]], row_bufs[b], sg[b]).wait()

        def start_out(j, b):
            off = base + j * C
            pltpu.async_copy(row_bufs[b], out_hbm.at[pl.ds(off, C)], so[b])

        def wait_out(j, b):
            off = base + j * C
            pltpu.make_async_copy(
                row_bufs[b], out_hbm.at[pl.ds(off, C)], so[b]).wait()

        for j in range(NBUF):
            start_gather(j, j)
            if j >= G:
                wait_gather(j - G)
                start_out(j - G, j - G)

        def body(kk, carry):
            for b in range(NBUF):
                j = kk * NBUF + b
                bg = (b - G) % NBUF
                wait_out(j - NBUF, b)
                start_gather(j, b)
                wait_gather(bg)
                start_out(j - G, bg)
            return carry

        lax.fori_loop(1, n_pass, body, 0)

        for m in range(n_chunks - G, n_chunks):
            wait_gather(m % NBUF)
            start_out(m, m % NBUF)
        for m in range(n_chunks - NBUF, n_chunks):
            wait_out(m, m % NBUF)

    return k


@jax.jit
def kernel(atom_types, table):
    n_mol, n_atom = atom_types.shape
    B = n_mol * n_atom
    flat = atom_types.reshape(B).astype(jnp.int32)
    out = _make_gather(B, table.shape[0])(flat, table)
    return out.reshape(n_mol, n_atom, EMB_D)
